# baseline (speedup 1.0000x reference)
import functools

import jax
import jax.numpy as jnp
from jax import lax
from jax.experimental import pallas as pl
from jax.experimental.pallas import tpu as pltpu
from jax.experimental.pallas import tpu_sc as plsc

N = 10000
E = 320000
NFEAT = 128
NHID = 128
NCLASS = 16

NC = 2
NS = 16
NW = NC * NS
CHUNK = 128
NCHUNK = 80
EPT = CHUNK * NCHUNK
E_PAD = EPT * NW
TRASH = 10008
ACC_ROWS = 10112
RPT = ACC_ROWS // NS
DEG_ROWS = 10240
DPT = DEG_ROWS // NS

ROW_BLK = 400

_mesh = plsc.VectorSubcoreMesh(core_axis_name="c", subcore_axis_name="s")


def _zero_f32(ref, nrow, ncol):
    z = jnp.zeros((16,), jnp.float32)

    def body(i, carry):
        for j in range(ncol // 16):
            ref[i, pl.ds(j * 16, 16)] = z
        return carry

    lax.fori_loop(0, nrow, body, 0)


def _stripe_copy(src, dst, src_base, dst_base):
    for j in range(4):
        pltpu.sync_copy(src.at[pl.ds(src_base + j * CHUNK, CHUNK)],
                        dst.at[pl.ds(dst_base + j * CHUNK, CHUNK)])
    pltpu.sync_copy(src.at[pl.ds(src_base + 512, RPT - 512)],
                    dst.at[pl.ds(dst_base + 512, RPT - 512)])


def _sc1_body(x_hbm, src_hbm, dst_hbm, aggA, aggB, degp,
              src2, dst2, rows_v, ones_v, dzero, acc, degacc, sem, semD):
    c = lax.axis_index("c")
    s = lax.axis_index("s")
    wid = c * NS + s

    pltpu.sync_copy(src_hbm.at[wid], src2)
    pltpu.sync_copy(dst_hbm.at[wid], dst2)

    _zero_f32(rows_v, CHUNK, NFEAT)
    z16 = jnp.zeros((16,), jnp.float32)
    for j in range(CHUNK // 16):
        ones_v[pl.ds(j * 16, 16)] = jnp.ones((16,), jnp.float32)

    def zd(i, carry):
        dzero[pl.ds(i * 16, 16)] = z16
        return carry

    lax.fori_loop(0, DPT // 16, zd, 0)

    zb = s * RPT
    for j in range(4):
        pltpu.sync_copy(rows_v, acc.at[pl.ds(zb + j * CHUNK, CHUNK)])
    pltpu.sync_copy(rows_v.at[pl.ds(0, RPT - 512)],
                    acc.at[pl.ds(zb + 512, RPT - 512)])
    pltpu.sync_copy(dzero, degacc.at[pl.ds(s * DPT, DPT)])
    plsc.subcore_barrier()

    def chunk(i, carry):
        pltpu.async_copy(x_hbm.at[src2.at[i]], rows_v, sem).wait()
        pltpu.sync_copy(rows_v, acc.at[dst2.at[i]], add=True)
        pltpu.sync_copy(ones_v, degacc.at[dst2.at[i]], add=True)
        return carry

    lax.fori_loop(0, NCHUNK, chunk, 0)
    plsc.subcore_barrier()

    @pl.when(c == 0)
    def _():
        _stripe_copy(acc, aggA, zb, zb)

    @pl.when(c == 1)
    def _():
        _stripe_copy(acc, aggB, zb, zb)

    pltpu.sync_copy(degacc.at[pl.ds(s * DPT, DPT)], degp.at[wid])


@jax.jit
def _sc1(x, srcp, dstp):
    return pl.kernel(
        _sc1_body,
        out_type=[
            jax.ShapeDtypeStruct((ACC_ROWS, NFEAT), jnp.float32),
            jax.ShapeDtypeStruct((ACC_ROWS, NFEAT), jnp.float32),
            jax.ShapeDtypeStruct((NW, DPT), jnp.float32),
        ],
        mesh=_mesh,
        scratch_types=[
            pltpu.VMEM((NCHUNK, CHUNK), jnp.int32),
            pltpu.VMEM((NCHUNK, CHUNK), jnp.int32),
            pltpu.VMEM((CHUNK, NFEAT), jnp.float32),
            pltpu.VMEM((CHUNK,), jnp.float32),
            pltpu.VMEM((DPT,), jnp.float32),
            pltpu.VMEM_SHARED((ACC_ROWS, NFEAT), jnp.float32),
            pltpu.VMEM_SHARED((DEG_ROWS,), jnp.float32),
            pltpu.SemaphoreType.DMA,
            pltpu.SemaphoreType.DMA,
        ],
    )(x, srcp, dstp)


def _sc2_body(h_hbm, src_hbm, dst_hbm, aggA, aggB,
              src2, dst2, rows_v, acc, sem):
    c = lax.axis_index("c")
    s = lax.axis_index("s")
    wid = c * NS + s

    pltpu.sync_copy(src_hbm.at[wid], src2)
    pltpu.sync_copy(dst_hbm.at[wid], dst2)

    _zero_f32(rows_v, CHUNK, NHID)
    zb = s * RPT
    for j in range(4):
        pltpu.sync_copy(rows_v, acc.at[pl.ds(zb + j * CHUNK, CHUNK)])
    pltpu.sync_copy(rows_v.at[pl.ds(0, RPT - 512)],
                    acc.at[pl.ds(zb + 512, RPT - 512)])
    plsc.subcore_barrier()

    def chunk(i, carry):
        pltpu.async_copy(h_hbm.at[src2.at[i]], rows_v, sem).wait()
        pltpu.sync_copy(rows_v, acc.at[dst2.at[i]], add=True)
        return carry

    lax.fori_loop(0, NCHUNK, chunk, 0)
    plsc.subcore_barrier()

    @pl.when(c == 0)
    def _():
        _stripe_copy(acc, aggA, zb, zb)

    @pl.when(c == 1)
    def _():
        _stripe_copy(acc, aggB, zb, zb)


@jax.jit
def _sc2(h, srcp, dstp):
    return pl.kernel(
        _sc2_body,
        out_type=[
            jax.ShapeDtypeStruct((ACC_ROWS, NHID), jnp.float32),
            jax.ShapeDtypeStruct((ACC_ROWS, NHID), jnp.float32),
        ],
        mesh=_mesh,
        scratch_types=[
            pltpu.VMEM((NCHUNK, CHUNK), jnp.int32),
            pltpu.VMEM((NCHUNK, CHUNK), jnp.int32),
            pltpu.VMEM((CHUNK, NHID), jnp.float32),
            pltpu.VMEM_SHARED((ACC_ROWS, NHID), jnp.float32),
            pltpu.SemaphoreType.DMA,
        ],
    )(h, srcp, dstp)


def _dense1_body(aggA_ref, aggB_ref, deg_ref, x_ref, w1l_ref, b1l_ref,
                 w1r_ref, w2r_ref, h_ref, hr_ref):
    deg = jnp.maximum(deg_ref[...], 1.0)
    mean = (aggA_ref[...] + aggB_ref[...]) / deg
    t1 = lax.dot_general(mean, w1l_ref[...], (((1,), (1,)), ((), ())),
                         preferred_element_type=jnp.float32)
    t2 = lax.dot_general(x_ref[...], w1r_ref[...], (((1,), (1,)), ((), ())),
                         preferred_element_type=jnp.float32)
    h = jnp.maximum(t1 + b1l_ref[...] + t2, 0.0)
    h_ref[...] = h
    hr_ref[...] = lax.dot_general(h, w2r_ref[...], (((1,), (1,)), ((), ())),
                                  preferred_element_type=jnp.float32)


@jax.jit
def _dense1(aggA, aggB, deg, x, W1l, b1l, W1r, W2r):
    nblk = N // ROW_BLK
    row_spec = pl.BlockSpec((ROW_BLK, NFEAT), lambda i: (i, 0))
    full = lambda shape: pl.BlockSpec(shape, lambda i: (0,) * len(shape))
    return pl.pallas_call(
        _dense1_body,
        grid=(nblk,),
        in_specs=[
            row_spec,
            row_spec,
            pl.BlockSpec((ROW_BLK, 1), lambda i: (i, 0)),
            row_spec,
            full((NHID, NFEAT)),
            full((1, NHID)),
            full((NHID, NFEAT)),
            full((NCLASS, NHID)),
        ],
        out_specs=[
            pl.BlockSpec((ROW_BLK, NHID), lambda i: (i, 0)),
            pl.BlockSpec((ROW_BLK, NCLASS), lambda i: (i, 0)),
        ],
        out_shape=[
            jax.ShapeDtypeStruct((N, NHID), jnp.float32),
            jax.ShapeDtypeStruct((N, NCLASS), jnp.float32),
        ],
    )(aggA, aggB, deg, x, W1l, b1l, W1r, W2r)


def _dense2_body(aggA_ref, aggB_ref, deg_ref, hr_ref, w2l_ref, b2l_ref,
                 out_ref):
    deg = jnp.maximum(deg_ref[...], 1.0)
    mean2 = (aggA_ref[...] + aggB_ref[...]) / deg
    t = lax.dot_general(mean2, w2l_ref[...], (((1,), (1,)), ((), ())),
                        preferred_element_type=jnp.float32)
    logits = t + b2l_ref[...] + hr_ref[...]
    m = jnp.max(logits, axis=1, keepdims=True)
    sumexp = jnp.sum(jnp.exp(logits - m), axis=1, keepdims=True)
    out_ref[...] = logits - m - jnp.log(sumexp)


@jax.jit
def _dense2(aggA, aggB, deg, hr, W2l, b2l):
    nblk = N // ROW_BLK
    spec16 = pl.BlockSpec((ROW_BLK, NCLASS), lambda i: (i, 0))
    spec128 = pl.BlockSpec((ROW_BLK, NHID), lambda i: (i, 0))
    return pl.pallas_call(
        _dense2_body,
        grid=(nblk,),
        in_specs=[
            spec128,
            spec128,
            pl.BlockSpec((ROW_BLK, 1), lambda i: (i, 0)),
            spec16,
            pl.BlockSpec((NCLASS, NHID), lambda i: (0, 0)),
            pl.BlockSpec((1, NCLASS), lambda i: (0, 0)),
        ],
        out_specs=spec16,
        out_shape=jax.ShapeDtypeStruct((N, NCLASS), jnp.float32),
    )(aggA, aggB, deg, hr, W2l, b2l)


def kernel(x, edge_index, W1l, b1l, W1r, W2l, b2l, W2r):
    src = edge_index[0].astype(jnp.int32)
    dst = edge_index[1].astype(jnp.int32)
    npad = E_PAD - E
    srcp = jnp.concatenate([src, jnp.zeros((npad,), jnp.int32)])
    dstp = jnp.concatenate([dst, jnp.full((npad,), TRASH, jnp.int32)])
    srcp = srcp.reshape(NW, NCHUNK, CHUNK)
    dstp = dstp.reshape(NW, NCHUNK, CHUNK)

    aggA, aggB, degp = _sc1(x, srcp, dstp)
    deg = (degp[:NS].reshape(-1)[:N] + degp[NS:].reshape(-1)[:N]).reshape(N, 1)
    h, hr = _dense1(aggA, aggB, deg, x, W1l, b1l.reshape(1, NHID), W1r, W2r)
    agg2A, agg2B = _sc2(h, srcp, dstp)
    return _dense2(agg2A, agg2B, deg, hr, W2l, b2l.reshape(1, NCLASS))


# speedup vs baseline: 1.0022x; 1.0022x over previous
import functools

import jax
import jax.numpy as jnp
from jax import lax
from jax.experimental import pallas as pl
from jax.experimental.pallas import tpu as pltpu
from jax.experimental.pallas import tpu_sc as plsc

N = 10000
E = 320000
NFEAT = 128
NHID = 128
NCLASS = 16

NC = 2
NS = 16
NW = NC * NS
CHUNK = 128
NCHUNK = 80
EPT = CHUNK * NCHUNK
E_PAD = EPT * NW
TRASH = 10008
ACC_ROWS = 10112
RPT = ACC_ROWS // NS
DEG_ROWS = 10240
DPT = DEG_ROWS // NS

ROW_BLK = 400

_mesh = plsc.VectorSubcoreMesh(core_axis_name="c", subcore_axis_name="s")


def _zero_f32(ref, nrow, ncol):
    z = jnp.zeros((16,), jnp.float32)

    def body(i, carry):
        for j in range(ncol // 16):
            ref[i, pl.ds(j * 16, 16)] = z
        return carry

    lax.fori_loop(0, nrow, body, 0)


def _stripe_copy(src, dst, src_base, dst_base):
    for j in range(4):
        pltpu.sync_copy(src.at[pl.ds(src_base + j * CHUNK, CHUNK)],
                        dst.at[pl.ds(dst_base + j * CHUNK, CHUNK)])
    pltpu.sync_copy(src.at[pl.ds(src_base + 512, RPT - 512)],
                    dst.at[pl.ds(dst_base + 512, RPT - 512)])


def _sc1_body(x_hbm, src_hbm, dst_hbm, aggA, aggB, degp,
              srcA, dstA, srcB, dstB, rows_v, ones_v, dzero, acc, degacc,
              semG, semIA, semIB):
    c = lax.axis_index("c")
    s = lax.axis_index("s")
    wid = c * NS + s

    _zero_f32(rows_v, CHUNK, NFEAT)
    z16 = jnp.zeros((16,), jnp.float32)
    for j in range(CHUNK // 16):
        ones_v[pl.ds(j * 16, 16)] = jnp.ones((16,), jnp.float32)

    def zd(i, carry):
        dzero[pl.ds(i * 16, 16)] = z16
        return carry

    lax.fori_loop(0, DPT // 16, zd, 0)

    zb = s * RPT
    for j in range(4):
        pltpu.sync_copy(rows_v, acc.at[pl.ds(zb + j * CHUNK, CHUNK)])
    pltpu.sync_copy(rows_v.at[pl.ds(0, RPT - 512)],
                    acc.at[pl.ds(zb + 512, RPT - 512)])
    pltpu.sync_copy(dzero, degacc.at[pl.ds(s * DPT, DPT)])
    plsc.subcore_barrier()

    base = wid * EPT
    pltpu.async_copy(src_hbm.at[pl.ds(base, CHUNK)], srcA, semIA)
    pltpu.async_copy(dst_hbm.at[pl.ds(base, CHUNK)], dstA, semIA)

    def do_chunk(off_next, src_c, dst_c, src_n, dst_n, semI_c, semI_n, last):
        if not last:
            pltpu.async_copy(src_hbm.at[pl.ds(off_next, CHUNK)], src_n,
                             semI_n)
            pltpu.async_copy(dst_hbm.at[pl.ds(off_next, CHUNK)], dst_n,
                             semI_n)
        pltpu.make_async_copy(src_hbm.at[pl.ds(base, CHUNK)], src_c,
                              semI_c).wait()
        pltpu.make_async_copy(dst_hbm.at[pl.ds(base, CHUNK)], dst_c,
                              semI_c).wait()
        pltpu.async_copy(x_hbm.at[src_c], rows_v, semG).wait()
        pltpu.sync_copy(rows_v, acc.at[dst_c], add=True)
        pltpu.sync_copy(ones_v, degacc.at[dst_c], add=True)

    def pair(i, carry):
        c0 = 2 * i
        do_chunk(base + (c0 + 1) * CHUNK, srcA, dstA, srcB, dstB,
                 semIA, semIB, False)
        do_chunk(base + (c0 + 2) * CHUNK, srcB, dstB, srcA, dstA,
                 semIB, semIA, False)
        return carry

    lax.fori_loop(0, NCHUNK // 2 - 1, pair, 0)
    do_chunk(base + (NCHUNK - 1) * CHUNK, srcA, dstA, srcB, dstB,
             semIA, semIB, False)
    do_chunk(0, srcB, dstB, srcA, dstA, semIB, semIA, True)
    plsc.subcore_barrier()

    @pl.when(c == 0)
    def _():
        _stripe_copy(acc, aggA, zb, zb)

    @pl.when(c == 1)
    def _():
        _stripe_copy(acc, aggB, zb, zb)

    pltpu.sync_copy(degacc.at[pl.ds(s * DPT, DPT)], degp.at[wid])


@jax.jit
def _sc1(x, srcp, dstp):
    return pl.kernel(
        _sc1_body,
        out_type=[
            jax.ShapeDtypeStruct((ACC_ROWS, NFEAT), jnp.float32),
            jax.ShapeDtypeStruct((ACC_ROWS, NFEAT), jnp.float32),
            jax.ShapeDtypeStruct((NW, DPT), jnp.float32),
        ],
        mesh=_mesh,
        scratch_types=[
            pltpu.VMEM((CHUNK,), jnp.int32),
            pltpu.VMEM((CHUNK,), jnp.int32),
            pltpu.VMEM((CHUNK,), jnp.int32),
            pltpu.VMEM((CHUNK,), jnp.int32),
            pltpu.VMEM((CHUNK, NFEAT), jnp.float32),
            pltpu.VMEM((CHUNK,), jnp.float32),
            pltpu.VMEM((DPT,), jnp.float32),
            pltpu.VMEM_SHARED((ACC_ROWS, NFEAT), jnp.float32),
            pltpu.VMEM_SHARED((DEG_ROWS,), jnp.float32),
            pltpu.SemaphoreType.DMA,
            pltpu.SemaphoreType.DMA,
            pltpu.SemaphoreType.DMA,
        ],
    )(x, srcp, dstp)


def _sc2_body(h_hbm, src_hbm, dst_hbm, aggA, aggB,
              srcA, dstA, srcB, dstB, rows_v, acc, semG, semIA, semIB):
    c = lax.axis_index("c")
    s = lax.axis_index("s")
    wid = c * NS + s

    _zero_f32(rows_v, CHUNK, NHID)
    zb = s * RPT
    for j in range(4):
        pltpu.sync_copy(rows_v, acc.at[pl.ds(zb + j * CHUNK, CHUNK)])
    pltpu.sync_copy(rows_v.at[pl.ds(0, RPT - 512)],
                    acc.at[pl.ds(zb + 512, RPT - 512)])
    plsc.subcore_barrier()

    base = wid * EPT
    pltpu.async_copy(src_hbm.at[pl.ds(base, CHUNK)], srcA, semIA)
    pltpu.async_copy(dst_hbm.at[pl.ds(base, CHUNK)], dstA, semIA)

    def do_chunk(off_next, src_c, dst_c, src_n, dst_n, semI_c, semI_n, last):
        if not last:
            pltpu.async_copy(src_hbm.at[pl.ds(off_next, CHUNK)], src_n,
                             semI_n)
            pltpu.async_copy(dst_hbm.at[pl.ds(off_next, CHUNK)], dst_n,
                             semI_n)
        pltpu.make_async_copy(src_hbm.at[pl.ds(base, CHUNK)], src_c,
                              semI_c).wait()
        pltpu.make_async_copy(dst_hbm.at[pl.ds(base, CHUNK)], dst_c,
                              semI_c).wait()
        pltpu.async_copy(h_hbm.at[src_c], rows_v, semG).wait()
        pltpu.sync_copy(rows_v, acc.at[dst_c], add=True)

    def pair(i, carry):
        c0 = 2 * i
        do_chunk(base + (c0 + 1) * CHUNK, srcA, dstA, srcB, dstB,
                 semIA, semIB, False)
        do_chunk(base + (c0 + 2) * CHUNK, srcB, dstB, srcA, dstA,
                 semIB, semIA, False)
        return carry

    lax.fori_loop(0, NCHUNK // 2 - 1, pair, 0)
    do_chunk(base + (NCHUNK - 1) * CHUNK, srcA, dstA, srcB, dstB,
             semIA, semIB, False)
    do_chunk(0, srcB, dstB, srcA, dstA, semIB, semIA, True)
    plsc.subcore_barrier()

    @pl.when(c == 0)
    def _():
        _stripe_copy(acc, aggA, zb, zb)

    @pl.when(c == 1)
    def _():
        _stripe_copy(acc, aggB, zb, zb)


@jax.jit
def _sc2(h, srcp, dstp):
    return pl.kernel(
        _sc2_body,
        out_type=[
            jax.ShapeDtypeStruct((ACC_ROWS, NHID), jnp.float32),
            jax.ShapeDtypeStruct((ACC_ROWS, NHID), jnp.float32),
        ],
        mesh=_mesh,
        scratch_types=[
            pltpu.VMEM((CHUNK,), jnp.int32),
            pltpu.VMEM((CHUNK,), jnp.int32),
            pltpu.VMEM((CHUNK,), jnp.int32),
            pltpu.VMEM((CHUNK,), jnp.int32),
            pltpu.VMEM((CHUNK, NHID), jnp.float32),
            pltpu.VMEM_SHARED((ACC_ROWS, NHID), jnp.float32),
            pltpu.SemaphoreType.DMA,
            pltpu.SemaphoreType.DMA,
            pltpu.SemaphoreType.DMA,
        ],
    )(h, srcp, dstp)


def _dense1_body(aggA_ref, aggB_ref, deg_ref, x_ref, w1l_ref, b1l_ref,
                 w1r_ref, w2r_ref, h_ref, hr_ref):
    deg = jnp.maximum(deg_ref[...], 1.0)
    mean = (aggA_ref[...] + aggB_ref[...]) / deg
    t1 = lax.dot_general(mean, w1l_ref[...], (((1,), (1,)), ((), ())),
                         preferred_element_type=jnp.float32)
    t2 = lax.dot_general(x_ref[...], w1r_ref[...], (((1,), (1,)), ((), ())),
                         preferred_element_type=jnp.float32)
    h = jnp.maximum(t1 + b1l_ref[...] + t2, 0.0)
    h_ref[...] = h
    hr_ref[...] = lax.dot_general(h, w2r_ref[...], (((1,), (1,)), ((), ())),
                                  preferred_element_type=jnp.float32)


@jax.jit
def _dense1(aggA, aggB, deg, x, W1l, b1l, W1r, W2r):
    nblk = N // ROW_BLK
    row_spec = pl.BlockSpec((ROW_BLK, NFEAT), lambda i: (i, 0))
    full = lambda shape: pl.BlockSpec(shape, lambda i: (0,) * len(shape))
    return pl.pallas_call(
        _dense1_body,
        grid=(nblk,),
        in_specs=[
            row_spec,
            row_spec,
            pl.BlockSpec((ROW_BLK, 1), lambda i: (i, 0)),
            row_spec,
            full((NHID, NFEAT)),
            full((1, NHID)),
            full((NHID, NFEAT)),
            full((NCLASS, NHID)),
        ],
        out_specs=[
            pl.BlockSpec((ROW_BLK, NHID), lambda i: (i, 0)),
            pl.BlockSpec((ROW_BLK, NCLASS), lambda i: (i, 0)),
        ],
        out_shape=[
            jax.ShapeDtypeStruct((N, NHID), jnp.float32),
            jax.ShapeDtypeStruct((N, NCLASS), jnp.float32),
        ],
    )(aggA, aggB, deg, x, W1l, b1l, W1r, W2r)


def _dense2_body(aggA_ref, aggB_ref, deg_ref, hr_ref, w2l_ref, b2l_ref,
                 out_ref):
    deg = jnp.maximum(deg_ref[...], 1.0)
    mean2 = (aggA_ref[...] + aggB_ref[...]) / deg
    t = lax.dot_general(mean2, w2l_ref[...], (((1,), (1,)), ((), ())),
                        preferred_element_type=jnp.float32)
    logits = t + b2l_ref[...] + hr_ref[...]
    m = jnp.max(logits, axis=1, keepdims=True)
    sumexp = jnp.sum(jnp.exp(logits - m), axis=1, keepdims=True)
    out_ref[...] = logits - m - jnp.log(sumexp)


@jax.jit
def _dense2(aggA, aggB, deg, hr, W2l, b2l):
    nblk = N // ROW_BLK
    spec16 = pl.BlockSpec((ROW_BLK, NCLASS), lambda i: (i, 0))
    spec128 = pl.BlockSpec((ROW_BLK, NHID), lambda i: (i, 0))
    return pl.pallas_call(
        _dense2_body,
        grid=(nblk,),
        in_specs=[
            spec128,
            spec128,
            pl.BlockSpec((ROW_BLK, 1), lambda i: (i, 0)),
            spec16,
            pl.BlockSpec((NCLASS, NHID), lambda i: (0, 0)),
            pl.BlockSpec((1, NCLASS), lambda i: (0, 0)),
        ],
        out_specs=spec16,
        out_shape=jax.ShapeDtypeStruct((N, NCLASS), jnp.float32),
    )(aggA, aggB, deg, hr, W2l, b2l)


def kernel(x, edge_index, W1l, b1l, W1r, W2l, b2l, W2r):
    src = edge_index[0].astype(jnp.int32)
    dst = edge_index[1].astype(jnp.int32)
    npad = E_PAD - E
    srcp = jnp.concatenate([src, jnp.zeros((npad,), jnp.int32)])
    dstp = jnp.concatenate([dst, jnp.full((npad,), TRASH, jnp.int32)])

    aggA, aggB, degp = _sc1(x, srcp, dstp)
    deg = (degp[:NS].reshape(-1)[:N] + degp[NS:].reshape(-1)[:N]).reshape(N, 1)
    h, hr = _dense1(aggA, aggB, deg, x, W1l, b1l.reshape(1, NHID), W1r, W2r)
    agg2A, agg2B = _sc2(h, srcp, dstp)
    return _dense2(agg2A, agg2B, deg, hr, W2l, b2l.reshape(1, NCLASS))


# speedup vs baseline: 2.7563x; 2.7501x over previous
import functools

import jax
import jax.numpy as jnp
from jax import lax
from jax.experimental import pallas as pl
from jax.experimental.pallas import tpu as pltpu
from jax.experimental.pallas import tpu_sc as plsc

N = 10000
E = 320000
NFEAT = 128
NHID = 128
NCLASS = 16

NC = 2
NS = 16
NW = NC * NS
CHUNK = 128
NCHUNK = 80
EPT = CHUNK * NCHUNK
E_PAD = EPT * NW
TRASH = 10008
ACC_ROWS = 10112
RPT = ACC_ROWS // NS
DEG_ROWS = 10240
DPT = DEG_ROWS // NS

ROW_BLK = 400

_mesh = plsc.VectorSubcoreMesh(core_axis_name="c", subcore_axis_name="s")


def _zero_f32(ref, nrow, ncol):
    z = jnp.zeros((16,), jnp.float32)

    def body(i, carry):
        for j in range(ncol // 16):
            ref[i, pl.ds(j * 16, 16)] = z
        return carry

    lax.fori_loop(0, nrow, body, 0)


def _stripe_copy(src, dst, src_base, dst_base):
    for j in range(4):
        pltpu.sync_copy(src.at[pl.ds(src_base + j * CHUNK, CHUNK)],
                        dst.at[pl.ds(dst_base + j * CHUNK, CHUNK)])
    pltpu.sync_copy(src.at[pl.ds(src_base + 512, RPT - 512)],
                    dst.at[pl.ds(dst_base + 512, RPT - 512)])


def _sc1_body(x_hbm, src_hbm, dst_hbm, aggA, aggB, degp,
              srcA, dstA, srcB, dstB, rows_v, ones_v, dzero, acc, degacc,
              semG, semIA, semIB):
    c = lax.axis_index("c")
    s = lax.axis_index("s")
    wid = c * NS + s

    _zero_f32(rows_v, CHUNK, NFEAT)
    z16 = jnp.zeros((16,), jnp.float32)
    for j in range(CHUNK // 16):
        ones_v[pl.ds(j * 16, 16)] = jnp.ones((16,), jnp.float32)

    def zd(i, carry):
        dzero[pl.ds(i * 16, 16)] = z16
        return carry

    lax.fori_loop(0, DPT // 16, zd, 0)

    zb = s * RPT
    for j in range(4):
        pltpu.sync_copy(rows_v, acc.at[pl.ds(zb + j * CHUNK, CHUNK)])
    pltpu.sync_copy(rows_v.at[pl.ds(0, RPT - 512)],
                    acc.at[pl.ds(zb + 512, RPT - 512)])
    pltpu.sync_copy(dzero, degacc.at[pl.ds(s * DPT, DPT)])
    plsc.subcore_barrier()

    base = wid * EPT
    pltpu.async_copy(src_hbm.at[pl.ds(base, CHUNK)], srcA, semIA)
    pltpu.async_copy(dst_hbm.at[pl.ds(base, CHUNK)], dstA, semIA)

    def do_chunk(off_next, src_c, dst_c, src_n, dst_n, semI_c, semI_n, last):
        if not last:
            pltpu.async_copy(src_hbm.at[pl.ds(off_next, CHUNK)], src_n,
                             semI_n)
            pltpu.async_copy(dst_hbm.at[pl.ds(off_next, CHUNK)], dst_n,
                             semI_n)
        pltpu.make_async_copy(src_hbm.at[pl.ds(base, CHUNK)], src_c,
                              semI_c).wait()
        pltpu.make_async_copy(dst_hbm.at[pl.ds(base, CHUNK)], dst_c,
                              semI_c).wait()
        pltpu.async_copy(x_hbm.at[src_c], rows_v, semG).wait()
        pltpu.sync_copy(rows_v, acc.at[dst_c], add=True)
        pltpu.sync_copy(ones_v, degacc.at[dst_c], add=True)

    def pair(i, carry):
        c0 = 2 * i
        do_chunk(base + (c0 + 1) * CHUNK, srcA, dstA, srcB, dstB,
                 semIA, semIB, False)
        do_chunk(base + (c0 + 2) * CHUNK, srcB, dstB, srcA, dstA,
                 semIB, semIA, False)
        return carry

    lax.fori_loop(0, NCHUNK // 2 - 1, pair, 0)
    do_chunk(base + (NCHUNK - 1) * CHUNK, srcA, dstA, srcB, dstB,
             semIA, semIB, False)
    do_chunk(0, srcB, dstB, srcA, dstA, semIB, semIA, True)
    plsc.subcore_barrier()

    @pl.when(c == 0)
    def _():
        _stripe_copy(acc, aggA, zb, zb)

    @pl.when(c == 1)
    def _():
        _stripe_copy(acc, aggB, zb, zb)

    pltpu.sync_copy(degacc.at[pl.ds(s * DPT, DPT)], degp.at[wid])


@jax.jit
def _sc1(x, srcp, dstp):
    return pl.kernel(
        _sc1_body,
        out_type=[
            jax.ShapeDtypeStruct((ACC_ROWS, NFEAT), jnp.float32),
            jax.ShapeDtypeStruct((ACC_ROWS, NFEAT), jnp.float32),
            jax.ShapeDtypeStruct((NW, DPT), jnp.float32),
        ],
        mesh=_mesh,
        scratch_types=[
            pltpu.VMEM((CHUNK,), jnp.int32),
            pltpu.VMEM((CHUNK,), jnp.int32),
            pltpu.VMEM((CHUNK,), jnp.int32),
            pltpu.VMEM((CHUNK,), jnp.int32),
            pltpu.VMEM((CHUNK, NFEAT), jnp.float32),
            pltpu.VMEM((CHUNK,), jnp.float32),
            pltpu.VMEM((DPT,), jnp.float32),
            pltpu.VMEM_SHARED((ACC_ROWS, NFEAT), jnp.float32),
            pltpu.VMEM_SHARED((DEG_ROWS,), jnp.float32),
            pltpu.SemaphoreType.DMA,
            pltpu.SemaphoreType.DMA,
            pltpu.SemaphoreType.DMA,
        ],
    )(x, srcp, dstp)


def _sc2_body(h_hbm, src_hbm, dst_hbm, aggA, aggB,
              srcA, dstA, srcB, dstB, rows_v, acc, semG, semIA, semIB):
    c = lax.axis_index("c")
    s = lax.axis_index("s")
    wid = c * NS + s

    _zero_f32(rows_v, CHUNK, NHID)
    zb = s * RPT
    for j in range(4):
        pltpu.sync_copy(rows_v, acc.at[pl.ds(zb + j * CHUNK, CHUNK)])
    pltpu.sync_copy(rows_v.at[pl.ds(0, RPT - 512)],
                    acc.at[pl.ds(zb + 512, RPT - 512)])
    plsc.subcore_barrier()

    base = wid * EPT
    pltpu.async_copy(src_hbm.at[pl.ds(base, CHUNK)], srcA, semIA)
    pltpu.async_copy(dst_hbm.at[pl.ds(base, CHUNK)], dstA, semIA)

    def do_chunk(off_next, src_c, dst_c, src_n, dst_n, semI_c, semI_n, last):
        if not last:
            pltpu.async_copy(src_hbm.at[pl.ds(off_next, CHUNK)], src_n,
                             semI_n)
            pltpu.async_copy(dst_hbm.at[pl.ds(off_next, CHUNK)], dst_n,
                             semI_n)
        pltpu.make_async_copy(src_hbm.at[pl.ds(base, CHUNK)], src_c,
                              semI_c).wait()
        pltpu.make_async_copy(dst_hbm.at[pl.ds(base, CHUNK)], dst_c,
                              semI_c).wait()
        pltpu.async_copy(h_hbm.at[src_c], rows_v, semG).wait()
        pltpu.sync_copy(rows_v, acc.at[dst_c], add=True)

    def pair(i, carry):
        c0 = 2 * i
        do_chunk(base + (c0 + 1) * CHUNK, srcA, dstA, srcB, dstB,
                 semIA, semIB, False)
        do_chunk(base + (c0 + 2) * CHUNK, srcB, dstB, srcA, dstA,
                 semIB, semIA, False)
        return carry

    lax.fori_loop(0, NCHUNK // 2 - 1, pair, 0)
    do_chunk(base + (NCHUNK - 1) * CHUNK, srcA, dstA, srcB, dstB,
             semIA, semIB, False)
    do_chunk(0, srcB, dstB, srcA, dstA, semIB, semIA, True)
    plsc.subcore_barrier()

    @pl.when(c == 0)
    def _():
        _stripe_copy(acc, aggA, zb, zb)

    @pl.when(c == 1)
    def _():
        _stripe_copy(acc, aggB, zb, zb)


@jax.jit
def _sc2(h, srcp, dstp):
    return pl.kernel(
        _sc2_body,
        out_type=[
            jax.ShapeDtypeStruct((ACC_ROWS, NHID), jnp.float32),
            jax.ShapeDtypeStruct((ACC_ROWS, NHID), jnp.float32),
        ],
        mesh=_mesh,
        scratch_types=[
            pltpu.VMEM((CHUNK,), jnp.int32),
            pltpu.VMEM((CHUNK,), jnp.int32),
            pltpu.VMEM((CHUNK,), jnp.int32),
            pltpu.VMEM((CHUNK,), jnp.int32),
            pltpu.VMEM((CHUNK, NHID), jnp.float32),
            pltpu.VMEM_SHARED((ACC_ROWS, NHID), jnp.float32),
            pltpu.SemaphoreType.DMA,
            pltpu.SemaphoreType.DMA,
            pltpu.SemaphoreType.DMA,
        ],
    )(h, srcp, dstp)


def _dense1_body(aggA_ref, aggB_ref, deg_ref, x_ref, w1l_ref, b1l_ref,
                 w1r_ref, w2r_ref, h_ref, hr_ref):
    deg = jnp.maximum(deg_ref[...], 1.0)
    mean = (aggA_ref[...] + aggB_ref[...]) / deg
    t1 = lax.dot_general(mean, w1l_ref[...], (((1,), (1,)), ((), ())),
                         preferred_element_type=jnp.float32)
    t2 = lax.dot_general(x_ref[...], w1r_ref[...], (((1,), (1,)), ((), ())),
                         preferred_element_type=jnp.float32)
    h = jnp.maximum(t1 + b1l_ref[...] + t2, 0.0)
    h_ref[...] = h
    hr_ref[...] = lax.dot_general(h, w2r_ref[...], (((1,), (1,)), ((), ())),
                                  preferred_element_type=jnp.float32)


@jax.jit
def _dense1(aggA, aggB, deg, x, W1l, b1l, W1r, W2r):
    nblk = N // ROW_BLK
    row_spec = pl.BlockSpec((ROW_BLK, NFEAT), lambda i: (i, 0))
    full = lambda shape: pl.BlockSpec(shape, lambda i: (0,) * len(shape))
    return pl.pallas_call(
        _dense1_body,
        grid=(nblk,),
        in_specs=[
            row_spec,
            row_spec,
            pl.BlockSpec((ROW_BLK, 1), lambda i: (i, 0)),
            row_spec,
            full((NHID, NFEAT)),
            full((1, NHID)),
            full((NHID, NFEAT)),
            full((NCLASS, NHID)),
        ],
        out_specs=[
            pl.BlockSpec((ROW_BLK, NHID), lambda i: (i, 0)),
            pl.BlockSpec((ROW_BLK, NCLASS), lambda i: (i, 0)),
        ],
        out_shape=[
            jax.ShapeDtypeStruct((N, NHID), jnp.float32),
            jax.ShapeDtypeStruct((N, NCLASS), jnp.float32),
        ],
    )(aggA, aggB, deg, x, W1l, b1l, W1r, W2r)


def _dense2_body(aggA_ref, aggB_ref, deg_ref, hr_ref, w2l_ref, b2l_ref,
                 out_ref):
    deg = jnp.maximum(deg_ref[...], 1.0)
    mean2 = (aggA_ref[...] + aggB_ref[...]) / deg
    t = lax.dot_general(mean2, w2l_ref[...], (((1,), (1,)), ((), ())),
                        preferred_element_type=jnp.float32)
    logits = t + b2l_ref[...] + hr_ref[...]
    m = jnp.max(logits, axis=1, keepdims=True)
    sumexp = jnp.sum(jnp.exp(logits - m), axis=1, keepdims=True)
    out_ref[...] = logits - m - jnp.log(sumexp)


@jax.jit
def _dense2(aggA, aggB, deg, hr, W2l, b2l):
    nblk = N // ROW_BLK
    spec16 = pl.BlockSpec((ROW_BLK, NCLASS), lambda i: (i, 0))
    spec128 = pl.BlockSpec((ROW_BLK, NHID), lambda i: (i, 0))
    return pl.pallas_call(
        _dense2_body,
        grid=(nblk,),
        in_specs=[
            spec128,
            spec128,
            pl.BlockSpec((ROW_BLK, 1), lambda i: (i, 0)),
            spec16,
            pl.BlockSpec((NCLASS, NHID), lambda i: (0, 0)),
            pl.BlockSpec((1, NCLASS), lambda i: (0, 0)),
        ],
        out_specs=spec16,
        out_shape=jax.ShapeDtypeStruct((N, NCLASS), jnp.float32),
    )(aggA, aggB, deg, hr, W2l, b2l)


def kernel(x, edge_index, W1l, b1l, W1r, W2l, b2l, W2r):
    src = edge_index[0].astype(jnp.int32)
    dst = edge_index[1].astype(jnp.int32)
    npad = E_PAD - E
    pad_ids = jnp.arange(npad, dtype=jnp.int32)
    srcp = jnp.concatenate([src, pad_ids % N])
    dstp = jnp.concatenate([dst, N + pad_ids % (ACC_ROWS - N)])

    aggA, aggB, degp = _sc1(x, srcp, dstp)
    deg = (degp[:NS].reshape(-1)[:N] + degp[NS:].reshape(-1)[:N]).reshape(N, 1)
    h, hr = _dense1(aggA, aggB, deg, x, W1l, b1l.reshape(1, NHID), W1r, W2r)
    agg2A, agg2B = _sc2(h, srcp, dstp)
    return _dense2(agg2A, agg2B, deg, hr, W2l, b2l.reshape(1, NCLASS))
